# baseline (speedup 1.0000x reference)
import jax
import jax.numpy as jnp
from jax import lax
from jax.experimental import pallas as pl
from jax.experimental.pallas import tpu as pltpu

_V = 100
_VP = 112
_E = 8
_H = 100
_HM = 104
_HH = 112
_G = 4 * _HM
_NC = _HM // 8
_K1 = _VP + _HH
_K2 = 2 * _HH + 16
_T = 80
_BBLK = 2048
_BH = 512
_NHALF = 4
_UNROLL = 8

_GATE_SCALE = (0.5, 0.5, 1.0, 0.5)


def _gate_rows(w):
    in_dim = w.shape[1]
    w4 = w.reshape(4, _H, in_dim) * jnp.asarray(_GATE_SCALE, w.dtype)[:, None, None]
    w4 = jnp.pad(w4, ((0, 0), (0, _HM - _H), (0, 0)))
    w4 = w4.reshape(4, _NC, 8, in_dim).transpose(1, 0, 2, 3)
    return w4.reshape(_G, in_dim)


def _gate_bias(b):
    b4 = b.reshape(4, _H) * jnp.asarray(_GATE_SCALE, b.dtype)[:, None]
    b4 = jnp.pad(b4, ((0, 0), (0, _HM - _H)))
    b4 = b4.reshape(4, _NC, 8).transpose(1, 0, 2)
    return b4.reshape(_G, 1)


def _hpad(w):
    return jnp.pad(0.5 * w, ((0, 0), (0, _HH - _H)))


def _lstm_body(xt_ref, embt_ref, wih0_ref, b0_ref, wh0_ref, w2_ref,
               fcw_ref, fcb_ref, out_ref, w1_ref):
    f32 = jnp.float32
    bf16 = jnp.bfloat16

    m0 = jnp.dot(wih0_ref[...], embt_ref[...], preferred_element_type=f32)
    w1_ref[:, 0:_VP] = (m0 + b0_ref[...]).astype(bf16)
    w1_ref[:, _VP:_K1] = wh0_ref[...]

    iota_bf = lax.broadcasted_iota(jnp.int32, (_VP, _BH), 0).astype(bf16)
    ones16 = jnp.ones((16, _BH), bf16)
    zeros8 = jnp.zeros((8, _BH), f32)

    def layer(w_ref, xin, c3):
        g = jnp.dot(w_ref[...], xin, preferred_element_type=f32)
        t3 = jnp.tanh(g).reshape(_NC, 32, _BH)
        ti = t3[:, 0:8, :]
        tf = t3[:, 8:16, :]
        tg = t3[:, 16:24, :]
        to = t3[:, 24:32, :]
        c3 = 0.5 * ((1.0 + tf) * c3 + (1.0 + ti) * tg)
        h3 = (1.0 + to) * jnp.tanh(c3)
        hf = jnp.concatenate([h3.reshape(_HM, _BH), zeros8], axis=0)
        return hf.astype(bf16), c3

    def half(xt_h, st):
        h1, c1, h2, c2 = st
        oh = jnp.where(iota_bf == xt_h, bf16(1.0), bf16(0.0))
        h1, c1 = layer(w1_ref, jnp.concatenate([oh, h1], axis=0), c1)
        h2, c2 = layer(w2_ref, jnp.concatenate([h1, h2, ones16], axis=0), c2)
        return h1, c1, h2, c2

    def step(tt, carry):
        for k in range(_UNROLL):
            xt = xt_ref[tt * _UNROLL + k].astype(bf16)
            carry = tuple(
                half(xt[:, i * _BH:(i + 1) * _BH], st)
                for i, st in enumerate(carry))
        return carry

    zb = jnp.zeros((_HH, _BH), bf16)
    zf = jnp.zeros((_NC, 8, _BH), f32)
    z4 = (zb, zf, zb, zf)
    states = lax.fori_loop(0, _T // _UNROLL, step, (z4,) * _NHALF)
    fcw = fcw_ref[...]
    fcb = fcb_ref[...]
    for i, st in enumerate(states):
        out_ref[i * _BH:(i + 1) * _BH, :] = (
            lax.dot_general(st[2], fcw, (((0,), (0,)), ((), ())),
                            preferred_element_type=f32) + fcb)


def kernel(x, embed_W, W_ih0, W_hh0, b_ih0, b_hh0, W_ih1, W_hh1, b_ih1,
           b_hh1, fc_W, fc_b):
    f32 = jnp.float32
    bf16 = jnp.bfloat16
    batch = x.shape[0]
    xt = x.T.reshape(_T, 1, batch)
    embt = jnp.pad(embed_W.T, ((0, 0), (0, _VP - _V)))
    wih0 = _gate_rows(W_ih0)
    b0 = jnp.broadcast_to(_gate_bias(b_ih0 + b_hh0), (_G, _VP))
    wh0 = _hpad(_gate_rows(W_hh0)).astype(bf16)
    bias_block = jnp.pad(_gate_bias(b_ih1 + b_hh1), ((0, 0), (0, 15)))
    w2 = jnp.concatenate(
        [_hpad(_gate_rows(W_ih1)), _hpad(_gate_rows(W_hh1)), bias_block],
        axis=1).astype(bf16)
    fcw = jnp.pad(0.5 * fc_W.T, ((0, _HH - _H), (0, 0))).astype(bf16)
    fcb = fc_b.reshape(1, _V)

    return pl.pallas_call(
        _lstm_body,
        out_shape=jax.ShapeDtypeStruct((batch, _V), f32),
        grid=(batch // _BBLK,),
        in_specs=[
            pl.BlockSpec((_T, 1, _BBLK), lambda j: (0, 0, j)),
            pl.BlockSpec((_E, _VP), lambda j: (0, 0)),
            pl.BlockSpec((_G, _E), lambda j: (0, 0)),
            pl.BlockSpec((_G, _VP), lambda j: (0, 0)),
            pl.BlockSpec((_G, _HH), lambda j: (0, 0)),
            pl.BlockSpec((_G, _K2), lambda j: (0, 0)),
            pl.BlockSpec((_HH, _V), lambda j: (0, 0)),
            pl.BlockSpec((1, _V), lambda j: (0, 0)),
        ],
        out_specs=pl.BlockSpec((_BBLK, _V), lambda j: (j, 0)),
        scratch_shapes=[pltpu.VMEM((_G, _K1), bf16)],
        compiler_params=pltpu.CompilerParams(
            dimension_semantics=("arbitrary",),
            vmem_limit_bytes=100 * 1024 * 1024,
        ),
    )(xt, embt, wih0, b0, wh0, w2, fcw, fcb)


# speedup vs baseline: 1.0160x; 1.0160x over previous
import jax
import jax.numpy as jnp
from jax import lax
from jax.experimental import pallas as pl
from jax.experimental.pallas import tpu as pltpu

_V = 100
_VP = 112
_E = 8
_H = 100
_HM = 104
_HH = 112
_G = 4 * _HM
_NC = _HM // 8
_K1 = _VP + _HH
_K2 = 2 * _HH + 16
_T = 80
_BBLK = 1024
_BH = 512
_NHALF = 2
_UNROLL = 16

_GATE_SCALE = (0.5, 0.5, 1.0, 0.5)


def _gate_rows(w):
    in_dim = w.shape[1]
    w4 = w.reshape(4, _H, in_dim) * jnp.asarray(_GATE_SCALE, w.dtype)[:, None, None]
    w4 = jnp.pad(w4, ((0, 0), (0, _HM - _H), (0, 0)))
    w4 = w4.reshape(4, _NC, 8, in_dim).transpose(1, 0, 2, 3)
    return w4.reshape(_G, in_dim)


def _gate_bias(b):
    b4 = b.reshape(4, _H) * jnp.asarray(_GATE_SCALE, b.dtype)[:, None]
    b4 = jnp.pad(b4, ((0, 0), (0, _HM - _H)))
    b4 = b4.reshape(4, _NC, 8).transpose(1, 0, 2)
    return b4.reshape(_G, 1)


def _hpad(w):
    return jnp.pad(0.5 * w, ((0, 0), (0, _HH - _H)))


def _lstm_body(xt_ref, embt_ref, wih0_ref, b0_ref, wh0_ref, w2_ref,
               fcw_ref, fcb_ref, out_ref, w1_ref):
    f32 = jnp.float32
    bf16 = jnp.bfloat16

    m0 = jnp.dot(wih0_ref[...], embt_ref[...], preferred_element_type=f32)
    w1_ref[:, 0:_VP] = (m0 + b0_ref[...]).astype(bf16)
    w1_ref[:, _VP:_K1] = wh0_ref[...]

    iota_bf = lax.broadcasted_iota(jnp.int32, (_VP, _BH), 0).astype(bf16)
    ones16 = jnp.ones((16, _BH), bf16)
    zeros8 = jnp.zeros((8, _BH), f32)

    def layer(w_ref, xin, c3):
        g = jnp.dot(w_ref[...], xin, preferred_element_type=f32)
        t3 = jnp.tanh(g).reshape(_NC, 32, _BH)
        ti = t3[:, 0:8, :]
        tf = t3[:, 8:16, :]
        tg = t3[:, 16:24, :]
        to = t3[:, 24:32, :]
        c3 = 0.5 * ((1.0 + tf) * c3 + (1.0 + ti) * tg)
        h3 = (1.0 + to) * jnp.tanh(c3)
        hf = jnp.concatenate([h3.reshape(_HM, _BH), zeros8], axis=0)
        return hf.astype(bf16), c3

    def half(xt_h, st):
        h1, c1, h2, c2 = st
        oh = jnp.where(iota_bf == xt_h, bf16(1.0), bf16(0.0))
        h1, c1 = layer(w1_ref, jnp.concatenate([oh, h1], axis=0), c1)
        h2, c2 = layer(w2_ref, jnp.concatenate([h1, h2, ones16], axis=0), c2)
        return h1, c1, h2, c2

    def step(tt, carry):
        for k in range(_UNROLL):
            xt = xt_ref[tt * _UNROLL + k].astype(bf16)
            carry = tuple(
                half(xt[:, i * _BH:(i + 1) * _BH], st)
                for i, st in enumerate(carry))
        return carry

    zb = jnp.zeros((_HH, _BH), bf16)
    zf = jnp.zeros((_NC, 8, _BH), f32)
    z4 = (zb, zf, zb, zf)
    states = lax.fori_loop(0, _T // _UNROLL, step, (z4,) * _NHALF)
    fcw = fcw_ref[...]
    fcb = fcb_ref[...]
    for i, st in enumerate(states):
        out_ref[i * _BH:(i + 1) * _BH, :] = (
            lax.dot_general(st[2], fcw, (((0,), (0,)), ((), ())),
                            preferred_element_type=f32) + fcb)


def kernel(x, embed_W, W_ih0, W_hh0, b_ih0, b_hh0, W_ih1, W_hh1, b_ih1,
           b_hh1, fc_W, fc_b):
    f32 = jnp.float32
    bf16 = jnp.bfloat16
    batch = x.shape[0]
    xt = x.T.reshape(_T, 1, batch)
    embt = jnp.pad(embed_W.T, ((0, 0), (0, _VP - _V)))
    wih0 = _gate_rows(W_ih0)
    b0 = jnp.broadcast_to(_gate_bias(b_ih0 + b_hh0), (_G, _VP))
    wh0 = _hpad(_gate_rows(W_hh0)).astype(bf16)
    bias_block = jnp.pad(_gate_bias(b_ih1 + b_hh1), ((0, 0), (0, 15)))
    w2 = jnp.concatenate(
        [_hpad(_gate_rows(W_ih1)), _hpad(_gate_rows(W_hh1)), bias_block],
        axis=1).astype(bf16)
    fcw = jnp.pad(0.5 * fc_W.T, ((0, _HH - _H), (0, 0))).astype(bf16)
    fcb = fc_b.reshape(1, _V)

    return pl.pallas_call(
        _lstm_body,
        out_shape=jax.ShapeDtypeStruct((batch, _V), f32),
        grid=(batch // _BBLK,),
        in_specs=[
            pl.BlockSpec((_T, 1, _BBLK), lambda j: (0, 0, j)),
            pl.BlockSpec((_E, _VP), lambda j: (0, 0)),
            pl.BlockSpec((_G, _E), lambda j: (0, 0)),
            pl.BlockSpec((_G, _VP), lambda j: (0, 0)),
            pl.BlockSpec((_G, _HH), lambda j: (0, 0)),
            pl.BlockSpec((_G, _K2), lambda j: (0, 0)),
            pl.BlockSpec((_HH, _V), lambda j: (0, 0)),
            pl.BlockSpec((1, _V), lambda j: (0, 0)),
        ],
        out_specs=pl.BlockSpec((_BBLK, _V), lambda j: (j, 0)),
        scratch_shapes=[pltpu.VMEM((_G, _K1), bf16)],
        compiler_params=pltpu.CompilerParams(
            dimension_semantics=("arbitrary",),
            vmem_limit_bytes=100 * 1024 * 1024,
            flags={"XLA_TPU_STORE_TO_LOAD_FORWARDING_WINDOW": 12288},
        ),
    )(xt, embt, wih0, b0, wh0, w2, fcw, fcb)


# speedup vs baseline: 1.0330x; 1.0168x over previous
import jax
import jax.numpy as jnp
from jax import lax
from jax.experimental import pallas as pl
from jax.experimental.pallas import tpu as pltpu

_V = 100
_VP = 112
_E = 8
_H = 100
_HM = 104
_HH = 112
_G = 4 * _HM
_NC = _HM // 8
_K1 = _VP + _HH
_K2 = 2 * _HH + 16
_T = 80
_BBLK = 1024
_BH = 512
_NHALF = 2
_UNROLL = 16

_GATE_SCALE = (0.5, 0.5, 1.0, 0.5)


def _gate_rows(w):
    in_dim = w.shape[1]
    w4 = w.reshape(4, _H, in_dim) * jnp.asarray(_GATE_SCALE, w.dtype)[:, None, None]
    w4 = jnp.pad(w4, ((0, 0), (0, _HM - _H), (0, 0)))
    w4 = w4.reshape(4, _NC, 8, in_dim).transpose(1, 0, 2, 3)
    return w4.reshape(_G, in_dim)


def _gate_bias(b):
    b4 = b.reshape(4, _H) * jnp.asarray(_GATE_SCALE, b.dtype)[:, None]
    b4 = jnp.pad(b4, ((0, 0), (0, _HM - _H)))
    b4 = b4.reshape(4, _NC, 8).transpose(1, 0, 2)
    return b4.reshape(_G, 1)


def _hpad(w):
    return jnp.pad(0.5 * w, ((0, 0), (0, _HH - _H)))


def _lstm_body(xt_ref, embt_ref, wih0_ref, b0_ref, wh0_ref, w2_ref,
               fcw_ref, fcb_ref, out_ref, w1_ref):
    f32 = jnp.float32
    bf16 = jnp.bfloat16

    m0 = jnp.dot(wih0_ref[...], embt_ref[...], preferred_element_type=f32)
    w1_ref[:, 0:_VP] = (m0 + b0_ref[...]).astype(bf16)
    w1_ref[:, _VP:_K1] = wh0_ref[...]

    iota_bf = lax.broadcasted_iota(jnp.int32, (_VP, _BH), 0).astype(bf16)
    ones16 = jnp.ones((16, _BH), bf16)
    zeros8 = jnp.zeros((8, _BH), f32)

    def layer(w_ref, xin, c3):
        g = jnp.dot(w_ref[...], xin, preferred_element_type=f32)
        t3 = jnp.tanh(g).reshape(_NC, 32, _BH)
        ti = t3[:, 0:8, :]
        tf = t3[:, 8:16, :]
        tg = t3[:, 16:24, :]
        to = t3[:, 24:32, :]
        c3 = 0.5 * ((1.0 + tf) * c3 + (1.0 + ti) * tg)
        h3 = (1.0 + to) * jnp.tanh(c3)
        hf = jnp.concatenate([h3.reshape(_HM, _BH), zeros8], axis=0)
        return hf.astype(bf16), c3

    def half(xt_h, st):
        h1, c1, h2, c2 = st
        oh = jnp.where(iota_bf == xt_h, bf16(1.0), bf16(0.0))
        h1, c1 = layer(w1_ref, jnp.concatenate([oh, h1], axis=0), c1)
        h2, c2 = layer(w2_ref, jnp.concatenate([h1, h2, ones16], axis=0), c2)
        return h1, c1, h2, c2

    def step(tt, carry):
        for k in range(_UNROLL):
            xt = xt_ref[tt * _UNROLL + k].astype(bf16)
            carry = tuple(
                half(xt[:, i * _BH:(i + 1) * _BH], st)
                for i, st in enumerate(carry))
        return carry

    zb = jnp.zeros((_HH, _BH), bf16)
    zf = jnp.zeros((_NC, 8, _BH), f32)
    z4 = (zb, zf, zb, zf)
    states = lax.fori_loop(0, _T // _UNROLL, step, (z4,) * _NHALF)
    fcw = fcw_ref[...]
    fcb = fcb_ref[...]
    for i, st in enumerate(states):
        out_ref[i * _BH:(i + 1) * _BH, :] = (
            lax.dot_general(st[2], fcw, (((0,), (0,)), ((), ())),
                            preferred_element_type=f32) + fcb)


def kernel(x, embed_W, W_ih0, W_hh0, b_ih0, b_hh0, W_ih1, W_hh1, b_ih1,
           b_hh1, fc_W, fc_b):
    f32 = jnp.float32
    bf16 = jnp.bfloat16
    batch = x.shape[0]
    xt = x.T.reshape(_T, 1, batch)
    embt = jnp.pad(embed_W.T, ((0, 0), (0, _VP - _V)))
    wih0 = _gate_rows(W_ih0)
    b0 = jnp.broadcast_to(_gate_bias(b_ih0 + b_hh0), (_G, _VP))
    wh0 = _hpad(_gate_rows(W_hh0)).astype(bf16)
    bias_block = jnp.pad(_gate_bias(b_ih1 + b_hh1), ((0, 0), (0, 15)))
    w2 = jnp.concatenate(
        [_hpad(_gate_rows(W_ih1)), _hpad(_gate_rows(W_hh1)), bias_block],
        axis=1).astype(bf16)
    fcw = jnp.pad(0.5 * fc_W.T, ((0, _HH - _H), (0, 0))).astype(bf16)
    fcb = fc_b.reshape(1, _V)

    return pl.pallas_call(
        _lstm_body,
        out_shape=jax.ShapeDtypeStruct((batch, _V), f32),
        grid=(batch // _BBLK,),
        in_specs=[
            pl.BlockSpec((_T, 1, _BBLK), lambda j: (0, 0, j)),
            pl.BlockSpec((_E, _VP), lambda j: (0, 0)),
            pl.BlockSpec((_G, _E), lambda j: (0, 0)),
            pl.BlockSpec((_G, _VP), lambda j: (0, 0)),
            pl.BlockSpec((_G, _HH), lambda j: (0, 0)),
            pl.BlockSpec((_G, _K2), lambda j: (0, 0)),
            pl.BlockSpec((_HH, _V), lambda j: (0, 0)),
            pl.BlockSpec((1, _V), lambda j: (0, 0)),
        ],
        out_specs=pl.BlockSpec((_BBLK, _V), lambda j: (j, 0)),
        scratch_shapes=[pltpu.VMEM((_G, _K1), bf16)],
        compiler_params=pltpu.CompilerParams(
            dimension_semantics=("arbitrary",),
            vmem_limit_bytes=100 * 1024 * 1024,
        ),
    )(xt, embt, wih0, b0, wh0, w2, fcw, fcb)
